# Initial kernel scaffold; baseline (speedup 1.0000x reference)
import functools

import jax
import jax.numpy as jnp
from jax import lax
from jax.experimental import pallas as pl
from jax.experimental.pallas import tpu as pltpu
from jax.experimental.pallas import tpu_sc as plsc

N = 10000
D = 128
GB = 128
GD = 3
LB = 256
LD = 2
NG = 64

NC = 2
NS = 16
NW = NC * NS
CH = 128

ET = 320000 + N
NCH = -(-ET // (NW * CH))
EP = NW * NCH * CH
NSEG_PAD = 10016

PNCH = -(-N // (NW * CH))
PEP = NW * PNCH * CH
PSEG_PAD = 80


def _spmm_builder(nch, nseg_pad, d):
  mesh = plsc.VectorSubcoreMesh(core_axis_name="c", subcore_axis_name="s")
  rows_per_tile = nseg_pad // NS
  co = []
  off = 0
  while off < rows_per_tile:
    sz = min(CH, rows_per_tile - off)
    co.append((off, sz))
    off += sz

  @functools.partial(
      pl.kernel,
      out_type=jax.ShapeDtypeStruct((NC, nseg_pad, d), jnp.float32),
      mesh=mesh,
      scratch_types=[
          pltpu.VMEM((nch, CH), jnp.int32),
          pltpu.VMEM((nch, CH), jnp.int32),
          pltpu.VMEM((CH, d), jnp.float32),
          pltpu.VMEM((CH, d), jnp.float32),
          pltpu.VMEM_SHARED((nseg_pad, d), jnp.float32),
          pltpu.SemaphoreType.DMA,
      ],
  )
  def spmm(src_hbm, dst_hbm, y_hbm, zeros_hbm, out_hbm,
           src_v, dst_v, rows_v, stage_v, acc_sh, gsem):
    cid = lax.axis_index("c")
    sid = lax.axis_index("s")
    tid = cid * NS + sid
    pltpu.sync_copy(zeros_hbm, stage_v)
    base = sid * rows_per_tile
    for off, sz in co:
      pltpu.sync_copy(stage_v.at[pl.ds(0, sz)],
                      acc_sh.at[pl.ds(base + off, sz)])
    pltpu.sync_copy(src_hbm.at[tid], src_v)
    pltpu.sync_copy(dst_hbm.at[tid], dst_v)
    plsc.subcore_barrier()

    def body(g, carry):
      pltpu.async_copy(y_hbm.at[src_v.at[g]], rows_v, gsem).wait()
      pltpu.sync_copy(rows_v, acc_sh.at[dst_v.at[g]], add=True)
      return carry

    lax.fori_loop(0, nch, body, 0)
    plsc.subcore_barrier()
    for off, sz in co:
      pltpu.sync_copy(acc_sh.at[pl.ds(base + off, sz)],
                      stage_v.at[pl.ds(0, sz)])
      pltpu.sync_copy(stage_v.at[pl.ds(0, sz)],
                      out_hbm.at[cid, pl.ds(base + off, sz)])

  return spmm


def _deg_builder(nch, nseg_pad):
  mesh = plsc.VectorSubcoreMesh(core_axis_name="c", subcore_axis_name="s")

  @functools.partial(
      pl.kernel,
      out_type=jax.ShapeDtypeStruct((NW, nseg_pad), jnp.float32),
      mesh=mesh,
      scratch_types=[
          pltpu.VMEM((nch, CH), jnp.int32),
          pltpu.VMEM((nseg_pad,), jnp.float32),
      ],
  )
  def degk(dst_hbm, out_hbm, dst_v, deg_v):
    cid = lax.axis_index("c")
    sid = lax.axis_index("s")
    tid = cid * NS + sid
    pltpu.sync_copy(dst_hbm.at[tid], dst_v)
    zeros16 = jnp.zeros((16,), jnp.float32)

    def zbody(i, carry):
      deg_v[pl.ds(i * 16, 16)] = zeros16
      return carry

    lax.fori_loop(0, nseg_pad // 16, zbody, 0)
    ones16 = jnp.ones((16,), jnp.float32)

    def body(g, carry):
      for t in range(CH // 16):
        idx = dst_v[g, pl.ds(t * 16, 16)]
        plsc.addupdate_scatter(deg_v, [idx], ones16)
      return carry

    lax.fori_loop(0, nch, body, 0)
    pltpu.sync_copy(deg_v, out_hbm.at[tid])

  return degk


_spmm_edges = _spmm_builder(NCH, NSEG_PAD, D)
_spmm_pool = _spmm_builder(PNCH, PSEG_PAD, D)
_deg_call = _deg_builder(NCH, NSEG_PAD)


def _dinv_body(degp_ref, out_ref):
  deg = jnp.sum(degp_ref[...], axis=0, keepdims=True)
  out_ref[...] = jnp.where(deg > 0, lax.rsqrt(jnp.maximum(deg, 1e-12)), 0.0)


def _u0_body(x_ref, w_ref, dinv_ref, u_ref):
  u_ref[...] = jnp.dot(x_ref[...], w_ref[...],
                       preferred_element_type=jnp.float32) * dinv_ref[...]


def _mid0_body(acc_ref, dinv_ref, b_ref, w_ref, u_ref):
  s = acc_ref[0, :N, :] + acc_ref[1, :N, :]
  h = jnp.maximum(s * dinv_ref[...] + b_ref[...], 0.0)
  u_ref[...] = jnp.dot(h, w_ref[...],
                       preferred_element_type=jnp.float32) * dinv_ref[...]


def _midbn_body(acc_ref, dinv_ref, b_ref, g_ref, bb_ref, w_ref, u_ref):
  t = (acc_ref[0, :N, :] + acc_ref[1, :N, :]) * dinv_ref[...] + b_ref[...]
  m = jnp.mean(t, axis=0, keepdims=True)
  v = jnp.mean((t - m) * (t - m), axis=0, keepdims=True)
  h = jnp.maximum((t - m) * lax.rsqrt(v + 1e-5) * g_ref[...] + bb_ref[...],
                  0.0)
  u_ref[...] = jnp.dot(h, w_ref[...],
                       preferred_element_type=jnp.float32) * dinv_ref[...]


def _finalh_body(acc_ref, dinv_ref, b_ref, g_ref, bb_ref, h_ref):
  t = (acc_ref[0, :N, :] + acc_ref[1, :N, :]) * dinv_ref[...] + b_ref[...]
  m = jnp.mean(t, axis=0, keepdims=True)
  v = jnp.mean((t - m) * (t - m), axis=0, keepdims=True)
  h_ref[...] = jnp.maximum(
      (t - m) * lax.rsqrt(v + 1e-5) * g_ref[...] + bb_ref[...], 0.0)


def _head_body(accp_ref, wl0_ref, bl0_ref, wl_ref, bl_ref, lng_ref, lnb_ref,
               wout_ref, bout_ref, out_ref):
  hp = accp_ref[0, :NG, :] + accp_ref[1, :NG, :]
  z = jnp.maximum(
      jnp.dot(hp, wl0_ref[...], preferred_element_type=jnp.float32)
      + bl0_ref[...], 0.0)
  for i in range(LD):
    z = jnp.dot(z, wl_ref[i], preferred_element_type=jnp.float32) + bl_ref[i]
    m = jnp.mean(z, axis=0, keepdims=True)
    v = jnp.mean((z - m) * (z - m), axis=0, keepdims=True)
    z = jnp.maximum(
        (z - m) * lax.rsqrt(v + 1e-5) * lng_ref[i][None, :]
        + lnb_ref[i][None, :], 0.0)
  out_ref[...] = jnp.dot(z, wout_ref[...],
                         preferred_element_type=jnp.float32) + bout_ref[...]


def _tc(body, out_shape, *args):
  return pl.pallas_call(
      body, out_shape=jax.ShapeDtypeStruct(out_shape, jnp.float32))(*args)


def kernel(x, edge_index, batch, W0, b0, Wg, bg, gng, gnb, Wl0, bl0, Wl, bl,
           lng, lnb, Wout, bout):
  loop = jnp.arange(N, dtype=edge_index.dtype)
  src = jnp.concatenate([edge_index[0], loop,
                         jnp.zeros((EP - ET,), edge_index.dtype)])
  dst = jnp.concatenate([edge_index[1], loop,
                         jnp.full((EP - ET,), N, edge_index.dtype)])
  src_r = src.reshape(NW, NCH, CH)
  dst_r = dst.reshape(NW, NCH, CH)
  zeros_blk = jnp.zeros((CH, D), jnp.float32)

  degp = _deg_call(dst_r)
  dinv_row = _tc(_dinv_body, (1, NSEG_PAD), degp)
  dinv_col = dinv_row[0, :N][:, None]

  u = _tc(_u0_body, (N, GB), x, W0, dinv_col)
  h = None
  for i in range(GD + 1):
    acc = _spmm_edges(src_r, dst_r, u, zeros_blk)
    if i == 0:
      u = _tc(_mid0_body, (N, GB), acc, dinv_col,
              b0.reshape(1, GB), Wg[0])
    elif i < GD:
      u = _tc(_midbn_body, (N, GB), acc, dinv_col,
              bg[i - 1].reshape(1, GB), gng[i - 1].reshape(1, GB),
              gnb[i - 1].reshape(1, GB), Wg[i])
    else:
      h = _tc(_finalh_body, (N, GB), acc, dinv_col,
              bg[GD - 1].reshape(1, GB), gng[GD - 1].reshape(1, GB),
              gnb[GD - 1].reshape(1, GB))

  psrc = jnp.concatenate(
      [loop, jnp.zeros((PEP - N,), edge_index.dtype)]).reshape(NW, PNCH, CH)
  pdst = jnp.concatenate(
      [batch, jnp.full((PEP - N,), NG, batch.dtype)]).reshape(NW, PNCH, CH)
  pacc = _spmm_pool(psrc, pdst, h, zeros_blk)

  out = _tc(_head_body, (NG, 2), pacc, Wl0, bl0.reshape(1, LB), Wl, bl,
            lng, lnb, Wout, bout.reshape(1, 2))
  return out


# speedup vs baseline: 7.6644x; 7.6644x over previous
import functools

import jax
import jax.numpy as jnp
from jax import lax
from jax.experimental import pallas as pl
from jax.experimental.pallas import tpu as pltpu
from jax.experimental.pallas import tpu_sc as plsc

N = 10000
D = 128
GB = 128
GD = 3
LB = 256
LD = 2
NG = 64

NC = 2
NS = 16
NW = NC * NS
CH = 128

ET = 320000 + N
NCH = -(-ET // (NW * CH))
EP = NW * NCH * CH
NSEG_PAD = 10240

PNCH = -(-N // (NW * CH))
PEP = NW * PNCH * CH
PSEG_PAD = 128


def _bcast_lane(v, j):
  return lax.gather(
      v, jnp.full((16, 1), j, jnp.int32),
      dimension_numbers=lax.GatherDimensionNumbers(
          offset_dims=(), collapsed_slice_dims=(0,), start_index_map=(0,)),
      slice_sizes=(1,), mode=lax.GatherScatterMode.PROMISE_IN_BOUNDS)


def _spmm_builder(nch, nseg_pad, d, scale):
  mesh = plsc.VectorSubcoreMesh(core_axis_name="c", subcore_axis_name="s")
  rows_per_tile = nseg_pad // NS
  co = []
  off = 0
  while off < rows_per_tile:
    sz = min(CH, rows_per_tile - off)
    co.append((off, sz))
    off += sz
  scratch = [
      pltpu.VMEM((nch, CH), jnp.int32),
      pltpu.VMEM((nch, CH), jnp.int32),
      pltpu.VMEM((CH, d), jnp.float32),
      pltpu.VMEM_SHARED((nseg_pad, d), jnp.float32),
      pltpu.SemaphoreType.DMA,
  ]
  if scale:
    scratch += [
        pltpu.VMEM((CH,), jnp.float32),
        pltpu.VMEM((CH,), jnp.float32),
        pltpu.SemaphoreType.DMA,
        pltpu.SemaphoreType.DMA,
    ]

  @functools.partial(
      pl.kernel,
      out_type=jax.ShapeDtypeStruct((NC, nseg_pad, d), jnp.float32),
      mesh=mesh,
      scratch_types=scratch,
  )
  def spmm(src_hbm, dst_hbm, y_hbm, zeros_hbm, dinv_hbm, out_hbm,
           src_v, dst_v, rows_v, acc_sh, gsem, *rest):
    cid = lax.axis_index("c")
    sid = lax.axis_index("s")
    tid = cid * NS + sid
    pltpu.sync_copy(zeros_hbm, rows_v)
    base = sid * rows_per_tile
    for off, sz in co:
      pltpu.sync_copy(rows_v.at[pl.ds(0, sz)],
                      acc_sh.at[pl.ds(base + off, sz)])
    pltpu.sync_copy(src_hbm.at[tid], src_v)
    pltpu.sync_copy(dst_hbm.at[tid], dst_v)
    plsc.subcore_barrier()

    def body(g, carry):
      rows_cp = pltpu.async_copy(y_hbm.at[src_v.at[g]], rows_v, gsem)
      if scale:
        dsrc_v, ddst_v, s1, s2 = rest
        ds_cp = pltpu.async_copy(dinv_hbm.at[src_v.at[g]], dsrc_v, s1)
        dd_cp = pltpu.async_copy(dinv_hbm.at[dst_v.at[g]], ddst_v, s2)
        ds_cp.wait()
        dd_cp.wait()
        rows_cp.wait()
        for t in range(CH // 16):
          sl16 = pl.ds(t * 16, 16)
          n16 = dsrc_v[sl16] * ddst_v[sl16]
          for j in range(16):
            nb = _bcast_lane(n16, j)
            e = t * 16 + j
            for q in range(d // 16):
              sl = pl.ds(q * 16, 16)
              rows_v[e, sl] = rows_v[e, sl] * nb
      else:
        rows_cp.wait()
      pltpu.sync_copy(rows_v, acc_sh.at[dst_v.at[g]], add=True)
      return carry

    lax.fori_loop(0, nch, body, 0)
    plsc.subcore_barrier()
    for off, sz in co:
      pltpu.sync_copy(acc_sh.at[pl.ds(base + off, sz)],
                      rows_v.at[pl.ds(0, sz)])
      pltpu.sync_copy(rows_v.at[pl.ds(0, sz)],
                      out_hbm.at[cid, pl.ds(base + off, sz)])

  return spmm


def _deg_builder(nch, nseg_pad):
  mesh = plsc.VectorSubcoreMesh(core_axis_name="c", subcore_axis_name="s")
  rows_per_tile = nseg_pad // NS

  @functools.partial(
      pl.kernel,
      out_type=jax.ShapeDtypeStruct((NC * nseg_pad,), jnp.float32),
      mesh=mesh,
      scratch_types=[
          pltpu.VMEM((nch, CH), jnp.int32),
          pltpu.VMEM((CH,), jnp.float32),
          pltpu.VMEM((rows_per_tile,), jnp.float32),
          pltpu.VMEM_SHARED((nseg_pad,), jnp.float32),
      ],
  )
  def degk(dst_hbm, ones_hbm, zeros_hbm, out_hbm, dst_v, ones_v, stage_v,
           deg_sh):
    cid = lax.axis_index("c")
    sid = lax.axis_index("s")
    tid = cid * NS + sid
    base = sid * rows_per_tile
    pltpu.sync_copy(zeros_hbm, stage_v)
    pltpu.sync_copy(stage_v, deg_sh.at[pl.ds(base, rows_per_tile)])
    pltpu.sync_copy(dst_hbm.at[tid], dst_v)
    pltpu.sync_copy(ones_hbm, ones_v)
    plsc.subcore_barrier()

    def body(g, carry):
      pltpu.sync_copy(ones_v, deg_sh.at[dst_v.at[g]], add=True)
      return carry

    lax.fori_loop(0, nch, body, 0)
    plsc.subcore_barrier()
    pltpu.sync_copy(deg_sh.at[pl.ds(base, rows_per_tile)], stage_v)
    pltpu.sync_copy(stage_v,
                    out_hbm.at[pl.ds(cid * nseg_pad + base, rows_per_tile)])

  return degk


_spmm_edges = _spmm_builder(NCH, NSEG_PAD, D, True)
_spmm_pool = _spmm_builder(PNCH, PSEG_PAD, D, False)
_deg_call = _deg_builder(NCH, NSEG_PAD)


def _mm(a, b):
  return jnp.dot(a.astype(jnp.bfloat16), b.astype(jnp.bfloat16),
                 preferred_element_type=jnp.float32)


def _dinv_body(degp_ref, out_ref):
  deg = jnp.sum(degp_ref[...], axis=0, keepdims=True)
  out_ref[...] = jnp.where(deg > 0, lax.rsqrt(jnp.maximum(deg, 1e-12)), 0.0)


def _u0_body(x_ref, w_ref, u_ref):
  u_ref[...] = _mm(x_ref[...], w_ref[...])


def _mm_body(a_ref, b_ref, o_ref):
  o_ref[...] = _mm(a_ref[...], b_ref[...])


def _tc(body, out_shape, *args):
  return pl.pallas_call(
      body, out_shape=jax.ShapeDtypeStruct(out_shape, jnp.float32))(*args)


def kernel(x, edge_index, batch, W0, b0, Wg, bg, gng, gnb, Wl0, bl0, Wl, bl,
           lng, lnb, Wout, bout):
  loop = jnp.arange(N, dtype=edge_index.dtype)
  src = jnp.concatenate([edge_index[0], loop,
                         jnp.zeros((EP - ET,), edge_index.dtype)])
  dst = jnp.concatenate([edge_index[1], loop,
                         jnp.full((EP - ET,), N, edge_index.dtype)])
  src_r = src.reshape(NW, NCH, CH)
  dst_r = dst.reshape(NW, NCH, CH)
  zeros_blk = jnp.zeros((CH, D), jnp.float32)

  ones_row = jnp.ones((CH,), jnp.float32)
  zeros_row = jnp.zeros((NSEG_PAD // NS,), jnp.float32)
  degp = _deg_call(dst_r, ones_row, zeros_row).reshape(NC, NSEG_PAD)
  dinv = _tc(_dinv_body, (1, NSEG_PAD), degp).reshape(NSEG_PAD)

  u = _tc(_u0_body, (N, GB), x, W0)
  h = None
  for i in range(GD + 1):
    acc = _spmm_edges(src_r, dst_r, u, zeros_blk, dinv)
    if i == 0:
      h = jax.nn.relu(acc[0, :N, :] + acc[1, :N, :] + b0)
    else:
      t = acc[0, :N, :] + acc[1, :N, :] + bg[i - 1]
      m = t.mean(axis=0)
      v = t.var(axis=0)
      h = jax.nn.relu((t - m) * lax.rsqrt(v + 1e-5) * gng[i - 1]
                      + gnb[i - 1])
    if i < GD:
      u = _tc(_mm_body, (N, GB), h, Wg[i])

  psrc = jnp.concatenate(
      [loop, jnp.zeros((PEP - N,), edge_index.dtype)]).reshape(NW, PNCH, CH)
  pdst = jnp.concatenate(
      [batch, jnp.full((PEP - N,), NG, batch.dtype)]).reshape(NW, PNCH, CH)
  pacc = _spmm_pool(psrc, pdst, h, zeros_blk, dinv)

  hp = pacc[0, :NG, :] + pacc[1, :NG, :]
  z = jax.nn.relu(_tc(_mm_body, (NG, LB), hp, Wl0) + bl0)
  for i in range(LD):
    z1 = _tc(_mm_body, (NG, LB), z, Wl[i]) + bl[i]
    m = z1.mean(axis=0)
    v = z1.var(axis=0)
    z = jax.nn.relu((z1 - m) * lax.rsqrt(v + 1e-5) * lng[i] + lnb[i])
  return _tc(_mm_body, (NG, 2), z, Wout) + bout
